# Initial kernel scaffold; baseline (speedup 1.0000x reference)
import jax
import jax.numpy as jnp
from jax.experimental import pallas as pl


def kernel(x, W):
    raise NotImplementedError("write your pallas kernel here")


# speedup vs baseline: 1.4902x; 1.4902x over previous
import numpy as np
import jax
import jax.numpy as jnp
from jax import lax
from jax.experimental import pallas as pl
from jax.experimental.pallas import tpu as pltpu
from jax.experimental.pallas import tpu_sc as plsc

_FIELD_DIMS = [38461] * 26
_F = 26
_D = 16
_H = 2
_B = 16384
_NC, _NS = 2, 16
_NW = _NC * _NS
_BPW = _B // _NW
_CB = 64
_NCHUNK = _BPW // _CB
_CBF = _CB * _F
_CBR = _CBF * _H
_NG = _CBR // 128

_off = np.concatenate([[0], np.cumsum(_FIELD_DIMS)[:-1]]).astype(np.int32)
_p = np.arange(_CBF)
_OFF_T = np.asarray(_off[_p % _F], dtype=np.int32)
_DMAP = np.asarray((_p // _F) * (_F * _H) + (_p % _F), dtype=np.int32)


def _body(w2, xf, offt, dmap, o, xv, offv, dmapv, idxv, bufv, sem):
    wid = lax.axis_index("s") * _NC + lax.axis_index("c")
    pltpu.sync_copy(offt, offv)
    pltpu.sync_copy(dmap, dmapv)

    for c in range(_NCHUNK):
        b0 = wid * _BPW + c * _CB
        pltpu.sync_copy(xf.at[pl.ds(b0 * _F, _CBF)], xv)

        def build(i, carry):
            s = pl.ds(pl.multiple_of(i * 16, 16), 16)
            e2 = (xv[s] + offv[s]) * 2
            d0 = dmapv[s]
            plsc.store_scatter(idxv, [d0], e2)
            plsc.store_scatter(idxv, [d0 + _F], e2 + 1)
            return carry

        lax.fori_loop(0, _CBF // 16, build, 0)

        copies = [
            pltpu.async_copy(w2.at[idxv.at[pl.ds(g * 128, 128)]],
                             bufv.at[pl.ds(g * 128, 128)], sem)
            for g in range(_NG)
        ]
        for cp in copies:
            cp.wait()
        pltpu.sync_copy(bufv, o.at[pl.ds(b0 * _F * _H, _CBR)])


_launch = pl.kernel(
    _body,
    out_type=jax.ShapeDtypeStruct((_B * _F * _H, _D), jnp.float32),
    mesh=plsc.VectorSubcoreMesh(core_axis_name="c", subcore_axis_name="s"),
    compiler_params=pltpu.CompilerParams(needs_layout_passes=False,
                                         use_tc_tiling_on_sc=False),
    scratch_types=[
        pltpu.VMEM((_CBF,), jnp.int32),
        pltpu.VMEM((_CBF,), jnp.int32),
        pltpu.VMEM((_CBF,), jnp.int32),
        pltpu.VMEM((_CBR,), jnp.int32),
        pltpu.VMEM((_CBR, _D), jnp.float32),
        pltpu.SemaphoreType.DMA,
    ],
)


@jax.jit
def kernel(x, W):
    w2 = W.reshape(-1, _D)
    xf = x.reshape(-1)
    o = _launch(w2, xf, jnp.asarray(_OFF_T), jnp.asarray(_DMAP))
    return o.reshape(_B, _H, _F * _D)


# speedup vs baseline: 1.4957x; 1.0037x over previous
import numpy as np
import jax
import jax.numpy as jnp
from jax import lax
from jax.experimental import pallas as pl
from jax.experimental.pallas import tpu as pltpu
from jax.experimental.pallas import tpu_sc as plsc

_FIELD_DIMS = [38461] * 26
_F = 26
_D = 16
_H = 2
_B = 16384
_NC, _NS = 2, 16
_NW = _NC * _NS
_BPW = _B // _NW
_CB = 64
_NCHUNK = _BPW // _CB
_CBF = _CB * _F
_NG = _CBF // 128

_off = np.concatenate([[0], np.cumsum(_FIELD_DIMS)[:-1]]).astype(np.int32)
_p = np.arange(_CBF)
_OFF_T = np.asarray(_off[_p % _F], dtype=np.int32)


def _body(w, xf, offt, o, xv, offv, idxv, bufr, bufo, sem):
    wid = lax.axis_index("s") * _NC + lax.axis_index("c")
    pltpu.sync_copy(offt, offv)

    for c in range(_NCHUNK):
        b0 = wid * _BPW + c * _CB
        pltpu.sync_copy(xf.at[pl.ds(b0 * _F, _CBF)], xv)

        def build(i, carry):
            s = pl.ds(pl.multiple_of(i * 16, 16), 16)
            idxv[s] = xv[s] + offv[s]
            return carry

        lax.fori_loop(0, _CBF // 16, build, 0)

        copies = [
            pltpu.async_copy(w.at[idxv.at[pl.ds(g * 128, 128)]],
                             bufr.at[pl.ds(g * 128, 128)], sem)
            for g in range(_NG)
        ]
        for cp in copies:
            cp.wait()

        def split(bb, carry):
            r = bb * _F
            for f in range(_F):
                bufo[bb, 0, pl.ds(f * _D, _D)] = bufr[r + f, pl.ds(0, _D)]
                bufo[bb, 1, pl.ds(f * _D, _D)] = bufr[r + f, pl.ds(_D, _D)]
            return carry

        lax.fori_loop(0, _CB, split, 0)
        pltpu.sync_copy(bufo, o.at[pl.ds(b0, _CB)])


_launch = pl.kernel(
    _body,
    out_type=jax.ShapeDtypeStruct((_B, _H, _F * _D), jnp.float32),
    mesh=plsc.VectorSubcoreMesh(core_axis_name="c", subcore_axis_name="s"),
    compiler_params=pltpu.CompilerParams(needs_layout_passes=False,
                                         use_tc_tiling_on_sc=False),
    scratch_types=[
        pltpu.VMEM((_CBF,), jnp.int32),
        pltpu.VMEM((_CBF,), jnp.int32),
        pltpu.VMEM((_CBF,), jnp.int32),
        pltpu.VMEM((_CBF, _H * _D), jnp.float32),
        pltpu.VMEM((_CB, _H, _F * _D), jnp.float32),
        pltpu.SemaphoreType.DMA,
    ],
)


@jax.jit
def kernel(x, W):
    return _launch(W, x.reshape(-1), jnp.asarray(_OFF_T))
